# baseline (speedup 1.0000x reference)
import functools
import jax
import jax.numpy as jnp
from jax import lax
from jax.experimental import pallas as pl
from jax.experimental.pallas import tpu as pltpu, tpu_sc as plsc

_CHUNK = 8
_STAGES = 3


def _make_sc(B, S, D):
    info = plsc.get_sparse_core_info()
    NC, NS, L = info.num_cores, info.num_subcores, info.num_lanes
    NW = NC * NS
    s_per_w = S // NW
    n_chunks = s_per_w // _CHUNK
    vregs = _CHUNK * (D // L)
    mesh = plsc.VectorSubcoreMesh(core_axis_name="c", subcore_axis_name="s")

    n_bufs = _STAGES * (B + 1)
    @functools.partial(
        pl.kernel,
        mesh=mesh,
        out_type=jax.ShapeDtypeStruct((B, S, D), jnp.float32),
        scratch_types=(
            [pltpu.VMEM((_CHUNK, D), jnp.float32)] * n_bufs
            + [pltpu.SemaphoreType.DMA] * (_STAGES * (B + 1) + _STAGES * B)
        ),
    )
    def k(x_hbm, pos_hbm, out_hbm, *scratch):
        bufs = scratch[:n_bufs]
        sems = scratch[n_bufs:]
        x_bufs = [[bufs[p * (B + 1) + b] for b in range(B)] for p in range(_STAGES)]
        pos_bufs = [bufs[p * (B + 1) + B] for p in range(_STAGES)]
        sem_x = [[sems[p * B + b] for b in range(B)] for p in range(_STAGES)]
        sem_p = [sems[_STAGES * B + p] for p in range(_STAGES)]
        sem_o = [[sems[_STAGES * (B + 1) + p * B + b] for b in range(B)]
                 for p in range(_STAGES)]

        wid = lax.axis_index("s") * NC + lax.axis_index("c")
        base = wid * s_per_w

        x_handles = [[None] * B for _ in range(_STAGES)]
        pos_handles = [None] * _STAGES
        out_handles = [[None] * B for _ in range(_STAGES)]

        def issue_loads(t):
            p = t % _STAGES
            sl = pl.ds(base + t * _CHUNK, _CHUNK)
            pos_handles[p] = pltpu.async_copy(pos_hbm.at[sl], pos_bufs[p],
                                              sem_p[p])
            for b in range(B):
                x_handles[p][b] = pltpu.async_copy(x_hbm.at[b, sl],
                                                   x_bufs[p][b], sem_x[p][b])

        issue_loads(0)
        if n_chunks > 1:
            issue_loads(1)

        for t in range(n_chunks):
            p = t % _STAGES

            pos_handles[p].wait()
            for b in range(B):
                x_handles[p][b].wait()

            xb, pb = x_bufs[p], pos_bufs[p]

            @plsc.parallel_loop(0, vregs, unroll=8)
            def _add(i, xb=xb, pb=pb):
                r = i // (D // L)
                sl = pl.ds((i % (D // L)) * L, L)
                pv = pb[r, sl]
                for b in range(B):
                    xb[b][r, sl] = xb[b][r, sl] + pv

            out_sl = pl.ds(base + t * _CHUNK, _CHUNK)
            for b in range(B):
                out_handles[p][b] = pltpu.async_copy(
                    xb[b], out_hbm.at[b, out_sl], sem_o[p][b])

            u = t + 2
            if u < n_chunks:
                q = u % _STAGES
                for b in range(B):
                    if out_handles[q][b] is not None:
                        out_handles[q][b].wait()
                        out_handles[q][b] = None
                issue_loads(u)

        for p in range(_STAGES):
            for b in range(B):
                if out_handles[p][b] is not None:
                    out_handles[p][b].wait()

    return k


def kernel(x, pos_table):
    B, S, D = x.shape
    pos = pos_table[:S]
    return _make_sc(B, S, D)(x, pos)


# speedup vs baseline: 1.0193x; 1.0193x over previous
import functools
import jax
import jax.numpy as jnp
from jax import lax
from jax.experimental import pallas as pl
from jax.experimental.pallas import tpu as pltpu, tpu_sc as plsc

_CHUNK = 16


def _make_sc(B, S, D):
    info = plsc.get_sparse_core_info()
    NC, NS, L = info.num_cores, info.num_subcores, info.num_lanes
    NW = NC * NS
    s_per_w = S // NW
    n_chunks = s_per_w // _CHUNK
    vregs_per_row = D // L
    n_steps = n_chunks * B
    mesh = plsc.VectorSubcoreMesh(core_axis_name="c", subcore_axis_name="s")

    @functools.partial(
        pl.kernel,
        mesh=mesh,
        out_type=jax.ShapeDtypeStruct((B, S, D), jnp.float32),
        scratch_types=(
            [pltpu.VMEM((_CHUNK, D), jnp.float32)] * 7
            + [pltpu.SemaphoreType.DMA] * 7
        ),
    )
    def k(x_hbm, pos_hbm, out_hbm,
          pos_v0, pos_v1, x_v0, x_v1, x_v2, o_v0, o_v1,
          sp0, sp1, sx0, sx1, sx2, so0, so1):
        wid = lax.axis_index("s") * NC + lax.axis_index("c")
        base = wid * s_per_w
        pos_bufs = [pos_v0, pos_v1]
        x_bufs = [x_v0, x_v1, x_v2]
        out_bufs = [o_v0, o_v1]
        sem_p, sem_x, sem_o = [sp0, sp1], [sx0, sx1, sx2], [so0, so1]

        x_handles = [None] * 3
        pos_handles = [None] * 2
        out_handles = [None] * 2

        def issue_x(s):
            t, b = divmod(s, B)
            xi = s % 3
            x_handles[xi] = pltpu.async_copy(
                x_hbm.at[b, pl.ds(base + t * _CHUNK, _CHUNK)],
                x_bufs[xi], sem_x[xi])

        def issue_pos(t):
            pp = t % 2
            pos_handles[pp] = pltpu.async_copy(
                pos_hbm.at[pl.ds(base + t * _CHUNK, _CHUNK)],
                pos_bufs[pp], sem_p[pp])

        issue_pos(0)
        for s0 in range(min(3, n_steps)):
            issue_x(s0)

        for s in range(n_steps):
            t, b = divmod(s, B)
            xi, oi, pp = s % 3, s % 2, t % 2

            x_handles[xi].wait()
            if b == 0:
                pos_handles[pp].wait()
            if out_handles[oi] is not None:
                out_handles[oi].wait()
                out_handles[oi] = None

            xb, pb, ob = x_bufs[xi], pos_bufs[pp], out_bufs[oi]

            @plsc.parallel_loop(0, _CHUNK * vregs_per_row, unroll=16)
            def _add(i, xb=xb, pb=pb, ob=ob):
                r = i // vregs_per_row
                sl = pl.ds((i % vregs_per_row) * L, L)
                ob[r, sl] = xb[r, sl] + pb[r, sl]

            out_handles[oi] = pltpu.async_copy(
                ob, out_hbm.at[b, pl.ds(base + t * _CHUNK, _CHUNK)],
                sem_o[oi])

            if b == 0 and t + 1 < n_chunks:
                issue_pos(t + 1)

            if s + 3 < n_steps:
                issue_x(s + 3)

        for oi in range(2):
            if out_handles[oi] is not None:
                out_handles[oi].wait()

    return k


def kernel(x, pos_table):
    B, S, D = x.shape
    pos = pos_table[:S]
    return _make_sc(B, S, D)(x, pos)


# speedup vs baseline: 1.0295x; 1.0100x over previous
import functools
import jax
import jax.numpy as jnp
from jax import lax
from jax.experimental import pallas as pl
from jax.experimental.pallas import tpu as pltpu, tpu_sc as plsc

_CHUNK = 16


def _make_sc(B, S, D):
    info = plsc.get_sparse_core_info()
    NC, NS, L = info.num_cores, info.num_subcores, info.num_lanes
    NW = NC * NS
    s_per_w = S // NW
    n_chunks = s_per_w // _CHUNK
    vregs_per_row = D // L
    n_steps = n_chunks * B
    mesh = plsc.VectorSubcoreMesh(core_axis_name="c", subcore_axis_name="s")

    @functools.partial(
        pl.kernel,
        mesh=mesh,
        out_type=jax.ShapeDtypeStruct((B, S, D), jnp.float32),
        scratch_types=(
            [pltpu.VMEM((_CHUNK, D), jnp.float32)] * 6
            + [pltpu.SemaphoreType.DMA] * 10
        ),
    )
    def k(x_hbm, pos_hbm, out_hbm,
          pos_v0, pos_v1, x_v0, x_v1, x_v2, x_v3,
          sp0, sp1, sx0, sx1, sx2, sx3, so0, so1, so2, so3):
        wid = lax.axis_index("s") * NC + lax.axis_index("c")
        base = wid * s_per_w
        pos_bufs = [pos_v0, pos_v1]
        x_bufs = [x_v0, x_v1, x_v2, x_v3]
        sem_p, sem_x = [sp0, sp1], [sx0, sx1, sx2, sx3]
        sem_o = [so0, so1, so2, so3]

        x_handles = [None] * 4
        pos_handles = [None] * 2
        out_handles = [None] * 4

        def issue_x(s):
            t, b = divmod(s, B)
            xi = s % 4
            x_handles[xi] = pltpu.async_copy(
                x_hbm.at[b, pl.ds(base + t * _CHUNK, _CHUNK)],
                x_bufs[xi], sem_x[xi])

        def issue_pos(t):
            pp = t % 2
            pos_handles[pp] = pltpu.async_copy(
                pos_hbm.at[pl.ds(base + t * _CHUNK, _CHUNK)],
                pos_bufs[pp], sem_p[pp])

        issue_pos(0)
        for s0 in range(min(3, n_steps)):
            issue_x(s0)

        for s in range(n_steps):
            t, b = divmod(s, B)
            xi, pp = s % 4, t % 2

            x_handles[xi].wait()
            if b == 0:
                pos_handles[pp].wait()

            xb, pb = x_bufs[xi], pos_bufs[pp]

            @plsc.parallel_loop(0, _CHUNK * vregs_per_row, unroll=8)
            def _add(i, xb=xb, pb=pb):
                r = i // vregs_per_row
                sl = pl.ds((i % vregs_per_row) * L, L)
                xb[r, sl] = xb[r, sl] + pb[r, sl]

            out_handles[xi] = pltpu.async_copy(
                xb, out_hbm.at[b, pl.ds(base + t * _CHUNK, _CHUNK)],
                sem_o[xi])

            if b == 0 and t + 1 < n_chunks:
                issue_pos(t + 1)

            u = s + 3
            if u < n_steps:
                q = u % 4
                if out_handles[q] is not None:
                    out_handles[q].wait()
                    out_handles[q] = None
                issue_x(u)

        for q in range(4):
            if out_handles[q] is not None:
                out_handles[q].wait()

    return k


def kernel(x, pos_table):
    B, S, D = x.shape
    pos = pos_table[:S]
    return _make_sc(B, S, D)(x, pos)


# speedup vs baseline: 1.1439x; 1.1111x over previous
import functools
import jax
import jax.numpy as jnp
from jax import lax
from jax.experimental import pallas as pl
from jax.experimental.pallas import tpu as pltpu, tpu_sc as plsc

_CHUNK = 16


def _make_sc(B, S, D):
    info = plsc.get_sparse_core_info()
    NC, NS, L = info.num_cores, info.num_subcores, info.num_lanes
    NW = NC * NS
    s_per_w = S // NW
    n_chunks = s_per_w // _CHUNK
    vregs_per_row = D // L
    n_steps = n_chunks * B
    mesh = plsc.VectorSubcoreMesh(core_axis_name="c", subcore_axis_name="s")

    @functools.partial(
        pl.kernel,
        mesh=mesh,
        out_type=jax.ShapeDtypeStruct((B, S, D), jnp.float32),
        scratch_types=(
            [pltpu.VMEM((_CHUNK, D), jnp.float32)] * 6
            + [pltpu.SemaphoreType.DMA] * 10
        ),
    )
    def k(x_hbm, pos_hbm, out_hbm,
          pos_v0, pos_v1, x_v0, x_v1, x_v2, x_v3,
          sp0, sp1, sx0, sx1, sx2, sx3, so0, so1, so2, so3):
        wid = lax.axis_index("s") * NC + lax.axis_index("c")
        base = wid * s_per_w
        pos_bufs = [pos_v0, pos_v1]
        x_bufs = [x_v0, x_v1, x_v2, x_v3]
        sem_p, sem_x = [sp0, sp1], [sx0, sx1, sx2, sx3]
        sem_o = [so0, so1, so2, so3]

        x_handles = [None] * 4
        pos_handles = [None] * 2
        out_handles = [None] * 4

        def issue_x(s):
            t, b = divmod(s, B)
            xi = s % 4
            x_handles[xi] = pltpu.async_copy(
                x_hbm.at[b, pl.ds(base + t * _CHUNK, _CHUNK)],
                x_bufs[xi], sem_x[xi])

        def issue_pos(t):
            pp = t % 2
            pos_handles[pp] = pltpu.async_copy(
                pos_hbm.at[pl.ds(base + t * _CHUNK, _CHUNK)],
                pos_bufs[pp], sem_p[pp])

        issue_pos(0)
        for s0 in range(min(3, n_steps)):
            issue_x(s0)

        for s in range(n_steps):
            t, b = divmod(s, B)
            xi, pp = s % 4, t % 2

            x_handles[xi].wait()
            if b == 0:
                pos_handles[pp].wait()

            xb, pb = x_bufs[xi], pos_bufs[pp]

            @plsc.parallel_loop(0, _CHUNK * vregs_per_row, unroll=8)
            def _add(i, xb=xb, pb=pb):
                r = i // vregs_per_row
                sl = pl.ds((i % vregs_per_row) * L, L)
                xb[r, sl] = xb[r, sl] + pb[r, sl]

            out_handles[xi] = pltpu.async_copy(
                xb, out_hbm.at[b, pl.ds(base + t * _CHUNK, _CHUNK)],
                sem_o[xi])

            if b == 0 and t + 1 < n_chunks:
                issue_pos(t + 1)

            u = s + 3
            if u < n_steps:
                q = u % 4
                if out_handles[q] is not None:
                    out_handles[q].wait()
                    out_handles[q] = None
                issue_x(u)

        for q in range(4):
            if out_handles[q] is not None:
                out_handles[q].wait()

    return k


def kernel(x, pos_table):
    B, S, D = x.shape
    return _make_sc(B, S, D)(x, pos_table)


# speedup vs baseline: 1.8840x; 1.6470x over previous
import jax
import jax.numpy as jnp
from jax.experimental import pallas as pl

_BLK = 2048


def _add_body(x_ref, pos_ref, o_ref):
    o_ref[...] = x_ref[...] + pos_ref[...][None]


def kernel(x, pos_table):
    B, S, D = x.shape
    grid = (S // _BLK, B)
    return pl.pallas_call(
        _add_body,
        grid=grid,
        in_specs=[
            pl.BlockSpec((1, _BLK, D), lambda i, b: (b, i, 0)),
            pl.BlockSpec((_BLK, D), lambda i, b: (i, 0)),
        ],
        out_specs=pl.BlockSpec((1, _BLK, D), lambda i, b: (b, i, 0)),
        out_shape=jax.ShapeDtypeStruct((B, S, D), x.dtype),
    )(x, pos_table)
